# baseline (speedup 1.0000x reference)
import functools

import jax
import jax.numpy as jnp
from jax import lax
from jax.experimental import pallas as pl
from jax.experimental.pallas import tpu as pltpu
from jax.experimental.pallas import tpu_sc as plsc

_N = 10000
_E = 320000
_H = 128
_L = 6
_NC = 2
_NS = 16
_NW = _NC * _NS
_RPT = 320
_NPAD = _NW * _RPT
_CH = 80
_EPAD = _E + 96
_RB = 1000
_GRID = _N // _RB

_mesh = plsc.VectorSubcoreMesh(core_axis_name="c", subcore_axis_name="s",
                               num_cores=_NC, num_subcores=_NS)

_f32 = jnp.float32
_i32 = jnp.int32


def _silu(v):
    return v / (1.0 + jnp.exp(-v))


def _edge_body(a_hbm, b_hbm, rec_hbm, srcp_hbm, dstp_hbm, lo_hbm, cnt_hbm,
               w_hbm, s_hbm, d_hbm,
               acc, dacc, rec_v, tmp_v, tmpd_v, idx_v, idxd_v, a_v, b_v, w_v,
               lo_v, cnt_v, sem, semd):
    w = lax.axis_index("s") * _NC + lax.axis_index("c")
    r0 = w * _RPT

    pltpu.sync_copy(w_hbm, w_v)
    pltpu.sync_copy(lo_hbm.at[w], lo_v)
    pltpu.sync_copy(cnt_hbm.at[w], cnt_v)
    lo = lo_v[:][0]
    cnt = cnt_v[:][0]

    def _z(i, _):
        for k in range(8):
            acc[i, pl.ds(k * 16, 16)] = jnp.zeros((16,), _f32)
        dacc[i, :] = jnp.zeros((16,), _f32)
        return 0
    lax.fori_loop(0, _RPT, _z, 0)

    nch = (cnt + _CH - 1) // _CH

    def _chunk(g, _):
        base = lo + g * _CH
        abase = (base // 8) * 8
        d = base - abase
        pltpu.sync_copy(rec_hbm.at[pl.ds(abase, 96)], rec_v)
        pltpu.sync_copy(srcp_hbm.at[pl.ds(abase, 96)], tmp_v)
        pltpu.sync_copy(dstp_hbm.at[pl.ds(abase, 96)], tmpd_v)
        for j in range(_CH // 16):
            idx_v[pl.ds(j * 16, 16)] = tmp_v[pl.ds(d + j * 16, 16)]
            idxd_v[pl.ds(j * 16, 16)] = tmpd_v[pl.ds(d + j * 16, 16)]
        cp_b = pltpu.async_copy(b_hbm.at[idx_v], b_v, sem)
        cp_a = pltpu.async_copy(a_hbm.at[idxd_v], a_v, semd)
        cp_b.wait()
        cp_a.wait()
        eend = jnp.minimum(cnt - g * _CH, _CH)

        def _edge(e, _):
            rec = rec_v[d + e, :]
            e0 = rec[0]
            e1 = rec[1]
            e2 = rec[2]
            ld = rec[3].astype(_i32) - r0
            for k in range(8):
                sl = pl.ds(k * 16, 16)
                pre = (a_v[e, sl] + b_v[e, sl] + w_v[3, sl]
                       + e0 * w_v[0, sl] + e1 * w_v[1, sl] + e2 * w_v[2, sl])
                acc[ld, sl] = acc[ld, sl] + pre / (1.0 + jnp.exp(-pre))
            dacc[ld, :] = dacc[ld, :] + 1.0
            return 0
        lax.fori_loop(0, eend, _edge, 0)
        return 0
    lax.fori_loop(0, nch, _chunk, 0)

    pltpu.sync_copy(acc, s_hbm.at[pl.ds(r0, _RPT)])
    pltpu.sync_copy(dacc, d_hbm.at[pl.ds(r0, _RPT)])


_edge_call = pl.kernel(
    _edge_body,
    out_type=(jax.ShapeDtypeStruct((_NPAD, _H), _f32),
              jax.ShapeDtypeStruct((_NPAD, 16), _f32)),
    mesh=_mesh,
    scratch_types=[
        pltpu.VMEM((_RPT, _H), _f32),
        pltpu.VMEM((_RPT, 16), _f32),
        pltpu.VMEM((96, 16), _f32),
        pltpu.VMEM((96,), _i32),
        pltpu.VMEM((96,), _i32),
        pltpu.VMEM((_CH,), _i32),
        pltpu.VMEM((_CH,), _i32),
        pltpu.VMEM((_CH, _H), _f32),
        pltpu.VMEM((_CH, _H), _f32),
        pltpu.VMEM((4, _H), _f32),
        pltpu.VMEM((16,), _i32),
        pltpu.VMEM((16,), _i32),
        pltpu.SemaphoreType.DMA,
        pltpu.SemaphoreType.DMA,
    ],
)


def _full(shape):
    return pl.BlockSpec(shape, lambda i: (0,) * len(shape))


def _rows(width):
    return pl.BlockSpec((_RB, width), lambda i: (i, 0))


def _enc_body(x_ref, w1_ref, b1_ref, w2_ref, b2_ref, o_ref):
    x = x_ref[...]
    w1 = w1_ref[...]
    h1 = x[:, 0:1] * w1[0:1, :] + x[:, 1:2] * w1[1:2, :] + b1_ref[...]
    h1 = _silu(h1)
    o_ref[...] = jnp.dot(h1, w2_ref[...], preferred_element_type=_f32) + b2_ref[...]


_enc_call = pl.pallas_call(
    _enc_body,
    grid=(_GRID,),
    in_specs=[_rows(2), _full((2, _H)), _full((1, _H)), _full((_H, _H)),
              _full((1, _H))],
    out_specs=_rows(_H),
    out_shape=jax.ShapeDtypeStruct((_N, _H), _f32),
)


def _ab_body(h_ref, wa_ref, wb_ref, a_ref, b_ref):
    h = h_ref[...]
    a_ref[...] = jnp.dot(h, wa_ref[...], preferred_element_type=_f32)
    b_ref[...] = jnp.dot(h, wb_ref[...], preferred_element_type=_f32)


_ab_call = pl.pallas_call(
    _ab_body,
    grid=(_GRID,),
    in_specs=[_rows(_H), _full((_H, _H)), _full((_H, _H))],
    out_specs=(pl.BlockSpec((_RB, _H), lambda i: (i, 0)),
               pl.BlockSpec((_RB, _H), lambda i: (i, 0))),
    out_shape=(jax.ShapeDtypeStruct((_NPAD, _H), _f32),
               jax.ShapeDtypeStruct((_N, _H), _f32)),
)


def _node_body(h_ref, s_ref, d_ref, p_ref, w2_ref, b2_ref, u1_ref, u2_ref,
               ub_ref, lng_ref, lnb_ref, fw1_ref, fb1_ref, fw2_ref, fb2_ref,
               o_ref):
    h = h_ref[...]
    deg = jnp.maximum(d_ref[:, 0:1], 1.0)
    agg = jnp.dot(s_ref[...] / deg, w2_ref[...],
                  preferred_element_type=_f32) + b2_ref[...]
    u = _silu(jnp.dot(h, u1_ref[...], preferred_element_type=_f32)
              + jnp.dot(agg, u2_ref[...], preferred_element_type=_f32)
              + ub_ref[...])
    mu = jnp.mean(u, axis=-1, keepdims=True)
    var = jnp.mean((u - mu) ** 2, axis=-1, keepdims=True)
    u = (u - mu) * lax.rsqrt(var + 1e-5) * lng_ref[...] + lnb_ref[...]
    f = _silu(jnp.dot(p_ref[...], fw1_ref[...], preferred_element_type=_f32)
              + fb1_ref[...])
    f = jnp.dot(f, fw2_ref[...], preferred_element_type=_f32) + fb2_ref[...]
    u = u * (1.0 + f[:, :_H]) + f[:, _H:]
    o_ref[...] = h + u


_node_call = pl.pallas_call(
    _node_body,
    grid=(_GRID,),
    in_specs=[
        _rows(_H),
        pl.BlockSpec((_RB, _H), lambda i: (i, 0)),
        pl.BlockSpec((_RB, 16), lambda i: (i, 0)),
        _rows(5),
        _full((_H, _H)), _full((1, _H)),
        _full((_H, _H)), _full((_H, _H)), _full((1, _H)),
        _full((1, _H)), _full((1, _H)),
        _full((5, _H)), _full((1, _H)), _full((_H, 2 * _H)), _full((1, 2 * _H)),
    ],
    out_specs=_rows(_H),
    out_shape=jax.ShapeDtypeStruct((_N, _H), _f32),
)


def _dec_body(h_ref, w1_ref, b1_ref, w2_ref, b2_ref, w3_ref, b3_ref, o_ref):
    t = _silu(jnp.dot(h_ref[...], w1_ref[...], preferred_element_type=_f32)
              + b1_ref[...])
    t = _silu(jnp.dot(t, w2_ref[...], preferred_element_type=_f32) + b2_ref[...])
    o_ref[...] = jnp.dot(t, w3_ref[...], preferred_element_type=_f32) + b3_ref[...]


_dec_call = pl.pallas_call(
    _dec_body,
    grid=(_GRID,),
    in_specs=[_rows(_H), _full((_H, _H)), _full((1, _H)),
              _full((_H, _H // 2)), _full((1, _H // 2)),
              _full((_H // 2, 8)), _full((1, 8))],
    out_specs=_rows(8),
    out_shape=jax.ShapeDtypeStruct((_N, 8), _f32),
)


def kernel(x, edge_index, edge_attr, params, enc_w1, enc_b1, enc_w2, enc_b2,
           msg_w1, msg_b1, msg_w2, msg_b2, upd_w, upd_b, ln_g, ln_b,
           film_w1, film_b1, film_w2, film_b2, dec_w1, dec_b1, dec_w2, dec_b2,
           dec_w3, dec_b3):
    src = edge_index[0]
    dst = edge_index[1]

    perm = jnp.argsort(dst)
    dst_s = dst[perm]
    rec = jnp.concatenate(
        [edge_attr[perm], dst_s[:, None].astype(_f32),
         jnp.zeros((_E, 12), _f32)], axis=1)
    rec = jnp.pad(rec, ((0, _EPAD - _E), (0, 0)))
    srcp = jnp.pad(src[perm], (0, _EPAD - _E))
    dstp = jnp.pad(dst_s, (0, _EPAD - _E))
    bounds = jnp.searchsorted(dst_s, jnp.arange(_NW + 1, dtype=_i32) * _RPT)
    bounds = bounds.astype(_i32)
    lo_t = jnp.broadcast_to(bounds[:_NW, None], (_NW, 16))
    cnt_t = jnp.broadcast_to((bounds[1:] - bounds[:_NW])[:, None], (_NW, 16))

    h = _enc_call(x, enc_w1, enc_b1.reshape(1, _H), enc_w2, enc_b2.reshape(1, _H))

    for l in range(_L):
        wa = msg_w1[l, :_H, :]
        wb = msg_w1[l, _H:2 * _H, :]
        wcb = jnp.concatenate([msg_w1[l, 2 * _H:, :], msg_b1[l][None, :]], axis=0)
        a_arr, b_arr = _ab_call(h, wa, wb)
        s_sum, degp = _edge_call(a_arr, b_arr, rec, srcp, dstp, lo_t, cnt_t, wcb)
        h = _node_call(h, s_sum, degp, params,
                       msg_w2[l], msg_b2[l].reshape(1, _H),
                       upd_w[l, :_H, :], upd_w[l, _H:, :],
                       upd_b[l].reshape(1, _H),
                       ln_g[l].reshape(1, _H), ln_b[l].reshape(1, _H),
                       film_w1[l], film_b1[l].reshape(1, _H),
                       film_w2[l], film_b2[l].reshape(1, 2 * _H))

    return _dec_call(h, dec_w1, dec_b1.reshape(1, _H),
                     dec_w2, dec_b2.reshape(1, _H // 2),
                     dec_w3, dec_b3.reshape(1, 8))


# speedup vs baseline: 1.2020x; 1.2020x over previous
import functools

import jax
import jax.numpy as jnp
from jax import lax
from jax.experimental import pallas as pl
from jax.experimental.pallas import tpu as pltpu
from jax.experimental.pallas import tpu_sc as plsc

_N = 10000
_E = 320000
_H = 128
_L = 6
_NC = 2
_NS = 16
_NW = _NC * _NS
_RPT = 320
_NPAD = _NW * _RPT
_CH = 80
_WIN = _CH + 16
_EPAD = _E + _WIN + 16
_RB = 1000
_GRID = _N // _RB

_mesh = plsc.VectorSubcoreMesh(core_axis_name="c", subcore_axis_name="s",
                               num_cores=_NC, num_subcores=_NS)

_f32 = jnp.float32
_i32 = jnp.int32


def _silu(v):
    return v / (1.0 + jnp.exp(-v))


def _edge_body(a_hbm, b_hbm, rec_hbm, srcp_hbm, dstp_hbm, lo_hbm, cnt_hbm,
               w_hbm, s_hbm,
               acc, rec2, tmpi, idxi, ab4, w_v, locnt,
               seml0, seml1, semg0, semg1):
    w = lax.axis_index("s") * _NC + lax.axis_index("c")
    r0 = w * _RPT
    semls = (seml0, seml1)
    semgs = (semg0, semg1)

    pltpu.sync_copy(w_hbm, w_v)
    pltpu.sync_copy(lo_hbm.at[w], locnt.at[0])
    pltpu.sync_copy(cnt_hbm.at[w], locnt.at[1])
    lo = locnt[0, :][0]
    cnt = locnt[1, :][0]
    nch = (cnt + _CH - 1) // _CH

    wk = [[w_v[r, pl.ds(k * 16, 16)] for k in range(8)] for r in range(4)]

    def _z(i, _):
        for k in range(8):
            acc[i, pl.ds(k * 16, 16)] = jnp.zeros((16,), _f32)
        return 0
    lax.fori_loop(0, _RPT, _z, 0)

    def _ab(g):
        base = lo + g * _CH
        abase = (base // 8) * 8
        return base, abase

    def fire_linear(g, s):
        _, ab = _ab(g)
        pltpu.async_copy(rec_hbm.at[pl.ds(ab, _WIN)], rec2.at[s], semls[s])
        pltpu.async_copy(srcp_hbm.at[pl.ds(ab, _WIN)], tmpi.at[s], semls[s])
        pltpu.async_copy(dstp_hbm.at[pl.ds(ab, _WIN)], tmpi.at[2 + s], semls[s])

    def wait_linear(g, s):
        _, ab = _ab(g)
        pltpu.make_async_copy(rec_hbm.at[pl.ds(ab, _WIN)], rec2.at[s], semls[s]).wait()
        pltpu.make_async_copy(srcp_hbm.at[pl.ds(ab, _WIN)], tmpi.at[s], semls[s]).wait()
        pltpu.make_async_copy(dstp_hbm.at[pl.ds(ab, _WIN)], tmpi.at[2 + s], semls[s]).wait()

    def build_fire_gather(g, s):
        base, ab = _ab(g)
        d = base - ab
        for j in range(_CH // 16):
            idxi[s, pl.ds(j * 16, 16)] = tmpi[s, pl.ds(d + j * 16, 16)]
            idxi[2 + s, pl.ds(j * 16, 16)] = tmpi[2 + s, pl.ds(d + j * 16, 16)]
        pltpu.async_copy(b_hbm.at[idxi.at[s]], ab4.at[s], semgs[s])
        pltpu.async_copy(a_hbm.at[idxi.at[2 + s]], ab4.at[2 + s], semgs[s])

    def wait_gather(s):
        pltpu.make_async_copy(b_hbm.at[idxi.at[s]], ab4.at[s], semgs[s]).wait()
        pltpu.make_async_copy(a_hbm.at[idxi.at[2 + s]], ab4.at[2 + s], semgs[s]).wait()

    def compute(g, s):
        base, ab = _ab(g)
        d = base - ab
        eend = jnp.minimum(cnt - g * _CH, _CH)

        def _edge(e, _):
            rec = rec2[s, d + e, :]
            e0 = rec[0]
            e1 = rec[1]
            e2 = rec[2]
            ld = rec[3].astype(_i32) - r0
            for k in range(8):
                sl = pl.ds(k * 16, 16)
                pre = (ab4[2 + s, e, sl] + ab4[s, e, sl] + wk[3][k]
                       + e0 * wk[0][k] + e1 * wk[1][k] + e2 * wk[2][k])
                acc[ld, sl] = acc[ld, sl] + pre / (1.0 + jnp.exp(-pre))
            return 0
        lax.fori_loop(0, eend, _edge, 0)

    @pl.when(nch > 0)
    def _():
        fire_linear(0, 0)
        wait_linear(0, 0)
        build_fire_gather(0, 0)

    @pl.when(nch > 1)
    def _():
        fire_linear(1, 1)

    def pair_body(gp, _):
        for sb in range(2):
            g = gp * 2 + sb

            @pl.when(g < nch)
            def _():
                wait_gather(sb)

                @pl.when(g + 1 < nch)
                def _():
                    wait_linear(g + 1, 1 - sb)
                    build_fire_gather(g + 1, 1 - sb)

                compute(g, sb)

                @pl.when(g + 2 < nch)
                def _():
                    fire_linear(g + 2, sb)
        return 0
    lax.fori_loop(0, (nch + 1) // 2, pair_body, 0)

    pltpu.sync_copy(acc, s_hbm.at[pl.ds(r0, _RPT)])


_edge_call = pl.kernel(
    _edge_body,
    out_type=jax.ShapeDtypeStruct((_NPAD, _H), _f32),
    mesh=_mesh,
    scratch_types=[
        pltpu.VMEM((_RPT, _H), _f32),
        pltpu.VMEM((2, _WIN, 16), _f32),
        pltpu.VMEM((4, _WIN), _i32),
        pltpu.VMEM((4, _CH), _i32),
        pltpu.VMEM((4, _CH, _H), _f32),
        pltpu.VMEM((4, _H), _f32),
        pltpu.VMEM((2, 16), _i32),
        pltpu.SemaphoreType.DMA,
        pltpu.SemaphoreType.DMA,
        pltpu.SemaphoreType.DMA,
        pltpu.SemaphoreType.DMA,
    ],
)


def _full(shape):
    return pl.BlockSpec(shape, lambda i: (0,) * len(shape))


def _rows(width):
    return pl.BlockSpec((_RB, width), lambda i: (i, 0))


def _enc_body(x_ref, w1_ref, b1_ref, w2_ref, b2_ref, o_ref):
    x = x_ref[...]
    w1 = w1_ref[...]
    h1 = x[:, 0:1] * w1[0:1, :] + x[:, 1:2] * w1[1:2, :] + b1_ref[...]
    h1 = _silu(h1)
    o_ref[...] = jnp.dot(h1, w2_ref[...], preferred_element_type=_f32) + b2_ref[...]


_enc_call = pl.pallas_call(
    _enc_body,
    grid=(_GRID,),
    in_specs=[_rows(2), _full((2, _H)), _full((1, _H)), _full((_H, _H)),
              _full((1, _H))],
    out_specs=_rows(_H),
    out_shape=jax.ShapeDtypeStruct((_N, _H), _f32),
)


def _ab_body(h_ref, wa_ref, wb_ref, a_ref, b_ref):
    h = h_ref[...]
    a_ref[...] = jnp.dot(h, wa_ref[...], preferred_element_type=_f32)
    b_ref[...] = jnp.dot(h, wb_ref[...], preferred_element_type=_f32)


_ab_call = pl.pallas_call(
    _ab_body,
    grid=(_GRID,),
    in_specs=[_rows(_H), _full((_H, _H)), _full((_H, _H))],
    out_specs=(pl.BlockSpec((_RB, _H), lambda i: (i, 0)),
               pl.BlockSpec((_RB, _H), lambda i: (i, 0))),
    out_shape=(jax.ShapeDtypeStruct((_NPAD, _H), _f32),
               jax.ShapeDtypeStruct((_N, _H), _f32)),
)


def _node_body(h_ref, s_ref, d_ref, p_ref, w2_ref, b2_ref, u1_ref, u2_ref,
               ub_ref, lng_ref, lnb_ref, fw1_ref, fb1_ref, fw2_ref, fb2_ref,
               o_ref):
    h = h_ref[...]
    deg = jnp.maximum(d_ref[...], 1.0)
    agg = jnp.dot(s_ref[...] / deg, w2_ref[...],
                  preferred_element_type=_f32) + b2_ref[...]
    u = _silu(jnp.dot(h, u1_ref[...], preferred_element_type=_f32)
              + jnp.dot(agg, u2_ref[...], preferred_element_type=_f32)
              + ub_ref[...])
    mu = jnp.mean(u, axis=-1, keepdims=True)
    var = jnp.mean((u - mu) ** 2, axis=-1, keepdims=True)
    u = (u - mu) * lax.rsqrt(var + 1e-5) * lng_ref[...] + lnb_ref[...]
    f = _silu(jnp.dot(p_ref[...], fw1_ref[...], preferred_element_type=_f32)
              + fb1_ref[...])
    f = jnp.dot(f, fw2_ref[...], preferred_element_type=_f32) + fb2_ref[...]
    u = u * (1.0 + f[:, :_H]) + f[:, _H:]
    o_ref[...] = h + u


_node_call = pl.pallas_call(
    _node_body,
    grid=(_GRID,),
    in_specs=[
        _rows(_H),
        pl.BlockSpec((_RB, _H), lambda i: (i, 0)),
        pl.BlockSpec((_RB, 1), lambda i: (i, 0)),
        _rows(5),
        _full((_H, _H)), _full((1, _H)),
        _full((_H, _H)), _full((_H, _H)), _full((1, _H)),
        _full((1, _H)), _full((1, _H)),
        _full((5, _H)), _full((1, _H)), _full((_H, 2 * _H)), _full((1, 2 * _H)),
    ],
    out_specs=_rows(_H),
    out_shape=jax.ShapeDtypeStruct((_N, _H), _f32),
)


def _dec_body(h_ref, w1_ref, b1_ref, w2_ref, b2_ref, w3_ref, b3_ref, o_ref):
    t = _silu(jnp.dot(h_ref[...], w1_ref[...], preferred_element_type=_f32)
              + b1_ref[...])
    t = _silu(jnp.dot(t, w2_ref[...], preferred_element_type=_f32) + b2_ref[...])
    o_ref[...] = jnp.dot(t, w3_ref[...], preferred_element_type=_f32) + b3_ref[...]


_dec_call = pl.pallas_call(
    _dec_body,
    grid=(_GRID,),
    in_specs=[_rows(_H), _full((_H, _H)), _full((1, _H)),
              _full((_H, _H // 2)), _full((1, _H // 2)),
              _full((_H // 2, 8)), _full((1, 8))],
    out_specs=_rows(8),
    out_shape=jax.ShapeDtypeStruct((_N, 8), _f32),
)


def kernel(x, edge_index, edge_attr, params, enc_w1, enc_b1, enc_w2, enc_b2,
           msg_w1, msg_b1, msg_w2, msg_b2, upd_w, upd_b, ln_g, ln_b,
           film_w1, film_b1, film_w2, film_b2, dec_w1, dec_b1, dec_w2, dec_b2,
           dec_w3, dec_b3):
    src = edge_index[0]
    dst = edge_index[1]

    perm = jnp.argsort(dst)
    dst_s = dst[perm]
    rec = jnp.concatenate(
        [edge_attr[perm], dst_s[:, None].astype(_f32),
         jnp.zeros((_E, 12), _f32)], axis=1)
    rec = jnp.pad(rec, ((0, _EPAD - _E), (0, 0)))
    srcp = jnp.pad(src[perm], (0, _EPAD - _E))
    dstp = jnp.pad(dst_s, (0, _EPAD - _E))
    bounds = jnp.searchsorted(dst_s, jnp.arange(_NW + 1, dtype=_i32) * _RPT)
    bounds = bounds.astype(_i32)
    lo_t = jnp.broadcast_to(bounds[:_NW, None], (_NW, 16))
    cnt_t = jnp.broadcast_to((bounds[1:] - bounds[:_NW])[:, None], (_NW, 16))
    nb = jnp.searchsorted(dst_s, jnp.arange(_N + 1, dtype=_i32))
    deg_arr = (nb[1:] - nb[:_N]).astype(_f32)[:, None]

    h = _enc_call(x, enc_w1, enc_b1.reshape(1, _H), enc_w2, enc_b2.reshape(1, _H))

    for l in range(_L):
        wa = msg_w1[l, :_H, :]
        wb = msg_w1[l, _H:2 * _H, :]
        wcb = jnp.concatenate([msg_w1[l, 2 * _H:, :], msg_b1[l][None, :]], axis=0)
        a_arr, b_arr = _ab_call(h, wa, wb)
        s_sum = _edge_call(a_arr, b_arr, rec, srcp, dstp, lo_t, cnt_t, wcb)
        h = _node_call(h, s_sum, deg_arr, params,
                       msg_w2[l], msg_b2[l].reshape(1, _H),
                       upd_w[l, :_H, :], upd_w[l, _H:, :],
                       upd_b[l].reshape(1, _H),
                       ln_g[l].reshape(1, _H), ln_b[l].reshape(1, _H),
                       film_w1[l], film_b1[l].reshape(1, _H),
                       film_w2[l], film_b2[l].reshape(1, 2 * _H))

    return _dec_call(h, dec_w1, dec_b1.reshape(1, _H),
                     dec_w2, dec_b2.reshape(1, _H // 2),
                     dec_w3, dec_b3.reshape(1, 8))


# speedup vs baseline: 1.2172x; 1.0127x over previous
import functools

import jax
import jax.numpy as jnp
from jax import lax
from jax.experimental import pallas as pl
from jax.experimental.pallas import tpu as pltpu
from jax.experimental.pallas import tpu_sc as plsc

_N = 10000
_E = 320000
_H = 128
_L = 6
_NC = 2
_NS = 16
_NW = _NC * _NS
_RPT = 320
_NPAD = _NW * _RPT
_CH = 112
_WIN = _CH + 8
_EPAD = _E + _WIN + 16
_RB = 1000
_GRID = _N // _RB

_mesh = plsc.VectorSubcoreMesh(core_axis_name="c", subcore_axis_name="s",
                               num_cores=_NC, num_subcores=_NS)

_f32 = jnp.float32
_i32 = jnp.int32


def _silu(v):
    return v / (1.0 + jnp.exp(-v))


def _edge_body(a_hbm, b_hbm, rec_hbm, srcp_hbm, dstp_hbm, lo_hbm, cnt_hbm,
               w_hbm, s_hbm,
               acc, rec2, tmpi, idxi, ab4, w_v, locnt,
               seml0, seml1, semg0, semg1):
    w = lax.axis_index("s") * _NC + lax.axis_index("c")
    r0 = w * _RPT
    semls = (seml0, seml1)
    semgs = (semg0, semg1)

    pltpu.sync_copy(w_hbm, w_v)
    pltpu.sync_copy(lo_hbm.at[w], locnt.at[0])
    pltpu.sync_copy(cnt_hbm.at[w], locnt.at[1])
    lo = locnt[0, :][0]
    cnt = locnt[1, :][0]
    nch = (cnt + _CH - 1) // _CH

    wk = [[w_v[r, pl.ds(k * 16, 16)] for k in range(8)] for r in range(4)]

    def _z(i, _):
        for k in range(8):
            acc[i, pl.ds(k * 16, 16)] = jnp.zeros((16,), _f32)
        return 0
    lax.fori_loop(0, _RPT, _z, 0)

    def _ab(g):
        base = lo + g * _CH
        abase = (base // 8) * 8
        return base, abase

    def fire_linear(g, s):
        _, ab = _ab(g)
        pltpu.async_copy(rec_hbm.at[pl.ds(ab, _WIN)], rec2.at[s], semls[s])
        pltpu.async_copy(srcp_hbm.at[pl.ds(ab, _WIN)], tmpi.at[s], semls[s])
        pltpu.async_copy(dstp_hbm.at[pl.ds(ab, _WIN)], tmpi.at[2 + s], semls[s])

    def wait_linear(g, s):
        _, ab = _ab(g)
        pltpu.make_async_copy(rec_hbm.at[pl.ds(ab, _WIN)], rec2.at[s], semls[s]).wait()
        pltpu.make_async_copy(srcp_hbm.at[pl.ds(ab, _WIN)], tmpi.at[s], semls[s]).wait()
        pltpu.make_async_copy(dstp_hbm.at[pl.ds(ab, _WIN)], tmpi.at[2 + s], semls[s]).wait()

    def build_fire_gather(g, s):
        base, ab = _ab(g)
        d = base - ab
        for j in range(_CH // 16):
            idxi[s, pl.ds(j * 16, 16)] = tmpi[s, pl.ds(d + j * 16, 16)]
            idxi[2 + s, pl.ds(j * 16, 16)] = tmpi[2 + s, pl.ds(d + j * 16, 16)]
        pltpu.async_copy(b_hbm.at[idxi.at[s]], ab4.at[s], semgs[s])
        pltpu.async_copy(a_hbm.at[idxi.at[2 + s]], ab4.at[2 + s], semgs[s])

    def wait_gather(s):
        pltpu.make_async_copy(b_hbm.at[idxi.at[s]], ab4.at[s], semgs[s]).wait()
        pltpu.make_async_copy(a_hbm.at[idxi.at[2 + s]], ab4.at[2 + s], semgs[s]).wait()

    def compute(g, s):
        base, ab = _ab(g)
        d = base - ab
        eend = jnp.minimum(cnt - g * _CH, _CH)

        def _edge(e, _):
            rec = rec2[s, d + e, :]
            e0 = rec[0]
            e1 = rec[1]
            e2 = rec[2]
            ld = rec[3].astype(_i32) - r0
            for k in range(8):
                sl = pl.ds(k * 16, 16)
                pre = (ab4[2 + s, e, sl] + ab4[s, e, sl] + wk[3][k]
                       + e0 * wk[0][k] + e1 * wk[1][k] + e2 * wk[2][k])
                acc[ld, sl] = acc[ld, sl] + pre / (1.0 + jnp.exp(-pre))
            return 0
        lax.fori_loop(0, eend, _edge, 0)

    @pl.when(nch > 0)
    def _():
        fire_linear(0, 0)
        wait_linear(0, 0)
        build_fire_gather(0, 0)

    @pl.when(nch > 1)
    def _():
        fire_linear(1, 1)

    def pair_body(gp, _):
        for sb in range(2):
            g = gp * 2 + sb

            @pl.when(g < nch)
            def _():
                wait_gather(sb)

                @pl.when(g + 1 < nch)
                def _():
                    wait_linear(g + 1, 1 - sb)
                    build_fire_gather(g + 1, 1 - sb)

                compute(g, sb)

                @pl.when(g + 2 < nch)
                def _():
                    fire_linear(g + 2, sb)
        return 0
    lax.fori_loop(0, (nch + 1) // 2, pair_body, 0)

    pltpu.sync_copy(acc, s_hbm.at[pl.ds(r0, _RPT)])


_edge_call = pl.kernel(
    _edge_body,
    out_type=jax.ShapeDtypeStruct((_NPAD, _H), _f32),
    mesh=_mesh,
    scratch_types=[
        pltpu.VMEM((_RPT, _H), _f32),
        pltpu.VMEM((2, _WIN, 16), _f32),
        pltpu.VMEM((4, _WIN), _i32),
        pltpu.VMEM((4, _CH), _i32),
        pltpu.VMEM((4, _CH, _H), _f32),
        pltpu.VMEM((4, _H), _f32),
        pltpu.VMEM((2, 16), _i32),
        pltpu.SemaphoreType.DMA,
        pltpu.SemaphoreType.DMA,
        pltpu.SemaphoreType.DMA,
        pltpu.SemaphoreType.DMA,
    ],
)


def _full(shape):
    return pl.BlockSpec(shape, lambda i: (0,) * len(shape))


def _rows(width):
    return pl.BlockSpec((_RB, width), lambda i: (i, 0))


def _enc_body(x_ref, w1_ref, b1_ref, w2_ref, b2_ref, o_ref):
    x = x_ref[...]
    w1 = w1_ref[...]
    h1 = x[:, 0:1] * w1[0:1, :] + x[:, 1:2] * w1[1:2, :] + b1_ref[...]
    h1 = _silu(h1)
    o_ref[...] = jnp.dot(h1, w2_ref[...], preferred_element_type=_f32) + b2_ref[...]


_enc_call = pl.pallas_call(
    _enc_body,
    grid=(_GRID,),
    in_specs=[_rows(2), _full((2, _H)), _full((1, _H)), _full((_H, _H)),
              _full((1, _H))],
    out_specs=_rows(_H),
    out_shape=jax.ShapeDtypeStruct((_N, _H), _f32),
)


def _ab_body(h_ref, wa_ref, wb_ref, a_ref, b_ref):
    h = h_ref[...]
    a_ref[...] = jnp.dot(h, wa_ref[...], preferred_element_type=_f32)
    b_ref[...] = jnp.dot(h, wb_ref[...], preferred_element_type=_f32)


_ab_call = pl.pallas_call(
    _ab_body,
    grid=(_GRID,),
    in_specs=[_rows(_H), _full((_H, _H)), _full((_H, _H))],
    out_specs=(pl.BlockSpec((_RB, _H), lambda i: (i, 0)),
               pl.BlockSpec((_RB, _H), lambda i: (i, 0))),
    out_shape=(jax.ShapeDtypeStruct((_NPAD, _H), _f32),
               jax.ShapeDtypeStruct((_N, _H), _f32)),
)


def _node_body(h_ref, s_ref, d_ref, p_ref, w2_ref, b2_ref, u1_ref, u2_ref,
               ub_ref, lng_ref, lnb_ref, fw1_ref, fb1_ref, fw2_ref, fb2_ref,
               o_ref):
    h = h_ref[...]
    deg = jnp.maximum(d_ref[...], 1.0)
    agg = jnp.dot(s_ref[...] / deg, w2_ref[...],
                  preferred_element_type=_f32) + b2_ref[...]
    u = _silu(jnp.dot(h, u1_ref[...], preferred_element_type=_f32)
              + jnp.dot(agg, u2_ref[...], preferred_element_type=_f32)
              + ub_ref[...])
    mu = jnp.mean(u, axis=-1, keepdims=True)
    var = jnp.mean((u - mu) ** 2, axis=-1, keepdims=True)
    u = (u - mu) * lax.rsqrt(var + 1e-5) * lng_ref[...] + lnb_ref[...]
    f = _silu(jnp.dot(p_ref[...], fw1_ref[...], preferred_element_type=_f32)
              + fb1_ref[...])
    f = jnp.dot(f, fw2_ref[...], preferred_element_type=_f32) + fb2_ref[...]
    u = u * (1.0 + f[:, :_H]) + f[:, _H:]
    o_ref[...] = h + u


_node_call = pl.pallas_call(
    _node_body,
    grid=(_GRID,),
    in_specs=[
        _rows(_H),
        pl.BlockSpec((_RB, _H), lambda i: (i, 0)),
        pl.BlockSpec((_RB, 1), lambda i: (i, 0)),
        _rows(5),
        _full((_H, _H)), _full((1, _H)),
        _full((_H, _H)), _full((_H, _H)), _full((1, _H)),
        _full((1, _H)), _full((1, _H)),
        _full((5, _H)), _full((1, _H)), _full((_H, 2 * _H)), _full((1, 2 * _H)),
    ],
    out_specs=_rows(_H),
    out_shape=jax.ShapeDtypeStruct((_N, _H), _f32),
)


def _dec_body(h_ref, w1_ref, b1_ref, w2_ref, b2_ref, w3_ref, b3_ref, o_ref):
    t = _silu(jnp.dot(h_ref[...], w1_ref[...], preferred_element_type=_f32)
              + b1_ref[...])
    t = _silu(jnp.dot(t, w2_ref[...], preferred_element_type=_f32) + b2_ref[...])
    o_ref[...] = jnp.dot(t, w3_ref[...], preferred_element_type=_f32) + b3_ref[...]


_dec_call = pl.pallas_call(
    _dec_body,
    grid=(_GRID,),
    in_specs=[_rows(_H), _full((_H, _H)), _full((1, _H)),
              _full((_H, _H // 2)), _full((1, _H // 2)),
              _full((_H // 2, 8)), _full((1, 8))],
    out_specs=_rows(8),
    out_shape=jax.ShapeDtypeStruct((_N, 8), _f32),
)


def kernel(x, edge_index, edge_attr, params, enc_w1, enc_b1, enc_w2, enc_b2,
           msg_w1, msg_b1, msg_w2, msg_b2, upd_w, upd_b, ln_g, ln_b,
           film_w1, film_b1, film_w2, film_b2, dec_w1, dec_b1, dec_w2, dec_b2,
           dec_w3, dec_b3):
    src = edge_index[0]
    dst = edge_index[1]

    perm = jnp.argsort(dst)
    dst_s = dst[perm]
    rec = jnp.concatenate(
        [edge_attr[perm], dst_s[:, None].astype(_f32),
         jnp.zeros((_E, 12), _f32)], axis=1)
    rec = jnp.pad(rec, ((0, _EPAD - _E), (0, 0)))
    srcp = jnp.pad(src[perm], (0, _EPAD - _E))
    dstp = jnp.pad(dst_s, (0, _EPAD - _E))
    bounds = jnp.searchsorted(dst_s, jnp.arange(_NW + 1, dtype=_i32) * _RPT)
    bounds = bounds.astype(_i32)
    lo_t = jnp.broadcast_to(bounds[:_NW, None], (_NW, 16))
    cnt_t = jnp.broadcast_to((bounds[1:] - bounds[:_NW])[:, None], (_NW, 16))
    nb = jnp.searchsorted(dst_s, jnp.arange(_N + 1, dtype=_i32))
    deg_arr = (nb[1:] - nb[:_N]).astype(_f32)[:, None]

    h = _enc_call(x, enc_w1, enc_b1.reshape(1, _H), enc_w2, enc_b2.reshape(1, _H))

    for l in range(_L):
        wa = msg_w1[l, :_H, :]
        wb = msg_w1[l, _H:2 * _H, :]
        wcb = jnp.concatenate([msg_w1[l, 2 * _H:, :], msg_b1[l][None, :]], axis=0)
        a_arr, b_arr = _ab_call(h, wa, wb)
        s_sum = _edge_call(a_arr, b_arr, rec, srcp, dstp, lo_t, cnt_t, wcb)
        h = _node_call(h, s_sum, deg_arr, params,
                       msg_w2[l], msg_b2[l].reshape(1, _H),
                       upd_w[l, :_H, :], upd_w[l, _H:, :],
                       upd_b[l].reshape(1, _H),
                       ln_g[l].reshape(1, _H), ln_b[l].reshape(1, _H),
                       film_w1[l], film_b1[l].reshape(1, _H),
                       film_w2[l], film_b2[l].reshape(1, 2 * _H))

    return _dec_call(h, dec_w1, dec_b1.reshape(1, _H),
                     dec_w2, dec_b2.reshape(1, _H // 2),
                     dec_w3, dec_b3.reshape(1, 8))


# speedup vs baseline: 2.7030x; 2.2206x over previous
import functools

import jax
import jax.numpy as jnp
from jax import lax
from jax.experimental import pallas as pl
from jax.experimental.pallas import tpu as pltpu
from jax.experimental.pallas import tpu_sc as plsc

_N = 10000
_E = 320000
_H = 128
_L = 6
_NC = 2
_NS = 16
_NW = _NC * _NS
_RPT = 320
_NPAD = _NW * _RPT
_CH = 112
_WIN = _CH + 8
_EPAD = _E + _WIN + 16
_RB = 1000
_GRID = _N // _RB

_mesh = plsc.VectorSubcoreMesh(core_axis_name="c", subcore_axis_name="s",
                               num_cores=_NC, num_subcores=_NS)

_f32 = jnp.float32
_i32 = jnp.int32


def _silu(v):
    return v / (1.0 + jnp.exp(-v))


def _edge_body(a_hbm, b_hbm, rec_hbm, srcp_hbm, dstp_hbm, lo_hbm, cnt_hbm,
               w_hbm, s_hbm,
               acc, rec2, tmpi, idxi, ab4, w_v, locnt,
               seml0, seml1, semg0, semg1):
    w = lax.axis_index("s") * _NC + lax.axis_index("c")
    r0 = w * _RPT
    semls = (seml0, seml1)
    semgs = (semg0, semg1)

    pltpu.sync_copy(w_hbm, w_v)
    pltpu.sync_copy(lo_hbm.at[w], locnt.at[0])
    pltpu.sync_copy(cnt_hbm.at[w], locnt.at[1])
    lo = locnt[0, :][0]
    cnt = locnt[1, :][0]
    nch = (cnt + _CH - 1) // _CH

    wk = [[w_v[r, pl.ds(k * 16, 16)] for k in range(8)] for r in range(4)]

    def _z(i, _):
        for k in range(8):
            acc[i, pl.ds(k * 16, 16)] = jnp.zeros((16,), _f32)
        return 0
    lax.fori_loop(0, _RPT, _z, 0)

    def _ab(g):
        base = lo + g * _CH
        abase = (base // 8) * 8
        return base, abase

    def fire_linear(g, s):
        _, ab = _ab(g)
        pltpu.async_copy(rec_hbm.at[pl.ds(ab, _WIN)], rec2.at[s], semls[s])
        pltpu.async_copy(srcp_hbm.at[pl.ds(ab, _WIN)], tmpi.at[s], semls[s])
        pltpu.async_copy(dstp_hbm.at[pl.ds(ab, _WIN)], tmpi.at[2 + s], semls[s])

    def wait_linear(g, s):
        _, ab = _ab(g)
        pltpu.make_async_copy(rec_hbm.at[pl.ds(ab, _WIN)], rec2.at[s], semls[s]).wait()
        pltpu.make_async_copy(srcp_hbm.at[pl.ds(ab, _WIN)], tmpi.at[s], semls[s]).wait()
        pltpu.make_async_copy(dstp_hbm.at[pl.ds(ab, _WIN)], tmpi.at[2 + s], semls[s]).wait()

    def build_fire_gather(g, s):
        base, ab = _ab(g)
        d = base - ab
        for j in range(_CH // 16):
            idxi[s, pl.ds(j * 16, 16)] = tmpi[s, pl.ds(d + j * 16, 16)]
            idxi[2 + s, pl.ds(j * 16, 16)] = tmpi[2 + s, pl.ds(d + j * 16, 16)]
        pltpu.async_copy(b_hbm.at[idxi.at[s]], ab4.at[s], semgs[s])
        pltpu.async_copy(a_hbm.at[idxi.at[2 + s]], ab4.at[2 + s], semgs[s])

    def wait_gather(s):
        pltpu.make_async_copy(b_hbm.at[idxi.at[s]], ab4.at[s], semgs[s]).wait()
        pltpu.make_async_copy(a_hbm.at[idxi.at[2 + s]], ab4.at[2 + s], semgs[s]).wait()

    def compute(g, s):
        base, ab = _ab(g)
        d = base - ab
        eend = jnp.minimum(cnt - g * _CH, _CH)

        def emit(e):
            rec = rec2[s, d + e, :]
            e0 = rec[0]
            e1 = rec[1]
            e2 = rec[2]
            ld = rec[3].astype(_i32) - r0
            vals = []
            for k in range(8):
                sl = pl.ds(k * 16, 16)
                pre = (ab4[2 + s, e, sl] + ab4[s, e, sl] + wk[3][k]
                       + e0 * wk[0][k] + e1 * wk[1][k] + e2 * wk[2][k])
                vals.append(pre / (1.0 + jnp.exp(-pre)))
            for k in range(8):
                sl = pl.ds(k * 16, 16)
                acc[ld, sl] = acc[ld, sl] + vals[k]

        def _edge2(p, _):
            emit(p * 2)
            emit(p * 2 + 1)
            return 0
        lax.fori_loop(0, eend // 2, _edge2, 0)

        @pl.when(eend % 2 == 1)
        def _():
            emit(eend - 1)

    @pl.when(nch > 0)
    def _():
        fire_linear(0, 0)
        wait_linear(0, 0)
        build_fire_gather(0, 0)

    @pl.when(nch > 1)
    def _():
        fire_linear(1, 1)

    def pair_body(gp, _):
        for sb in range(2):
            g = gp * 2 + sb

            @pl.when(g < nch)
            def _():
                wait_gather(sb)

                @pl.when(g + 1 < nch)
                def _():
                    wait_linear(g + 1, 1 - sb)
                    build_fire_gather(g + 1, 1 - sb)

                compute(g, sb)

                @pl.when(g + 2 < nch)
                def _():
                    fire_linear(g + 2, sb)
        return 0
    lax.fori_loop(0, (nch + 1) // 2, pair_body, 0)

    pltpu.sync_copy(acc, s_hbm.at[pl.ds(r0, _RPT)])


_edge_call = pl.kernel(
    _edge_body,
    out_type=jax.ShapeDtypeStruct((_NPAD, _H), _f32),
    mesh=_mesh,
    scratch_types=[
        pltpu.VMEM((_RPT, _H), _f32),
        pltpu.VMEM((2, _WIN, 16), _f32),
        pltpu.VMEM((4, _WIN), _i32),
        pltpu.VMEM((4, _CH), _i32),
        pltpu.VMEM((4, _CH, _H), _f32),
        pltpu.VMEM((4, _H), _f32),
        pltpu.VMEM((2, 16), _i32),
        pltpu.SemaphoreType.DMA,
        pltpu.SemaphoreType.DMA,
        pltpu.SemaphoreType.DMA,
        pltpu.SemaphoreType.DMA,
    ],
)


def _full(shape):
    return pl.BlockSpec(shape, lambda i: (0,) * len(shape))


def _rows(width):
    return pl.BlockSpec((_RB, width), lambda i: (i, 0))


def _enc_body(x_ref, w1_ref, b1_ref, w2_ref, b2_ref, o_ref):
    x = x_ref[...]
    w1 = w1_ref[...]
    h1 = x[:, 0:1] * w1[0:1, :] + x[:, 1:2] * w1[1:2, :] + b1_ref[...]
    h1 = _silu(h1)
    o_ref[...] = jnp.dot(h1, w2_ref[...], preferred_element_type=_f32) + b2_ref[...]


_enc_call = pl.pallas_call(
    _enc_body,
    grid=(_GRID,),
    in_specs=[_rows(2), _full((2, _H)), _full((1, _H)), _full((_H, _H)),
              _full((1, _H))],
    out_specs=_rows(_H),
    out_shape=jax.ShapeDtypeStruct((_N, _H), _f32),
)


def _ab_body(h_ref, wa_ref, wb_ref, a_ref, b_ref):
    h = h_ref[...]
    a_ref[...] = jnp.dot(h, wa_ref[...], preferred_element_type=_f32)
    b_ref[...] = jnp.dot(h, wb_ref[...], preferred_element_type=_f32)


_ab_call = pl.pallas_call(
    _ab_body,
    grid=(_GRID,),
    in_specs=[_rows(_H), _full((_H, _H)), _full((_H, _H))],
    out_specs=(pl.BlockSpec((_RB, _H), lambda i: (i, 0)),
               pl.BlockSpec((_RB, _H), lambda i: (i, 0))),
    out_shape=(jax.ShapeDtypeStruct((_NPAD, _H), _f32),
               jax.ShapeDtypeStruct((_N, _H), _f32)),
)


def _node_body(h_ref, s_ref, d_ref, p_ref, w2_ref, b2_ref, u1_ref, u2_ref,
               ub_ref, lng_ref, lnb_ref, fw1_ref, fb1_ref, fw2_ref, fb2_ref,
               o_ref):
    h = h_ref[...]
    deg = jnp.maximum(d_ref[...], 1.0)
    agg = jnp.dot(s_ref[...] / deg, w2_ref[...],
                  preferred_element_type=_f32) + b2_ref[...]
    u = _silu(jnp.dot(h, u1_ref[...], preferred_element_type=_f32)
              + jnp.dot(agg, u2_ref[...], preferred_element_type=_f32)
              + ub_ref[...])
    mu = jnp.mean(u, axis=-1, keepdims=True)
    var = jnp.mean((u - mu) ** 2, axis=-1, keepdims=True)
    u = (u - mu) * lax.rsqrt(var + 1e-5) * lng_ref[...] + lnb_ref[...]
    f = _silu(jnp.dot(p_ref[...], fw1_ref[...], preferred_element_type=_f32)
              + fb1_ref[...])
    f = jnp.dot(f, fw2_ref[...], preferred_element_type=_f32) + fb2_ref[...]
    u = u * (1.0 + f[:, :_H]) + f[:, _H:]
    o_ref[...] = h + u


_node_call = pl.pallas_call(
    _node_body,
    grid=(_GRID,),
    in_specs=[
        _rows(_H),
        pl.BlockSpec((_RB, _H), lambda i: (i, 0)),
        pl.BlockSpec((_RB, 1), lambda i: (i, 0)),
        _rows(5),
        _full((_H, _H)), _full((1, _H)),
        _full((_H, _H)), _full((_H, _H)), _full((1, _H)),
        _full((1, _H)), _full((1, _H)),
        _full((5, _H)), _full((1, _H)), _full((_H, 2 * _H)), _full((1, 2 * _H)),
    ],
    out_specs=_rows(_H),
    out_shape=jax.ShapeDtypeStruct((_N, _H), _f32),
)


def _dec_body(h_ref, w1_ref, b1_ref, w2_ref, b2_ref, w3_ref, b3_ref, o_ref):
    t = _silu(jnp.dot(h_ref[...], w1_ref[...], preferred_element_type=_f32)
              + b1_ref[...])
    t = _silu(jnp.dot(t, w2_ref[...], preferred_element_type=_f32) + b2_ref[...])
    o_ref[...] = jnp.dot(t, w3_ref[...], preferred_element_type=_f32) + b3_ref[...]


_dec_call = pl.pallas_call(
    _dec_body,
    grid=(_GRID,),
    in_specs=[_rows(_H), _full((_H, _H)), _full((1, _H)),
              _full((_H, _H // 2)), _full((1, _H // 2)),
              _full((_H // 2, 8)), _full((1, 8))],
    out_specs=_rows(8),
    out_shape=jax.ShapeDtypeStruct((_N, 8), _f32),
)


def kernel(x, edge_index, edge_attr, params, enc_w1, enc_b1, enc_w2, enc_b2,
           msg_w1, msg_b1, msg_w2, msg_b2, upd_w, upd_b, ln_g, ln_b,
           film_w1, film_b1, film_w2, film_b2, dec_w1, dec_b1, dec_w2, dec_b2,
           dec_w3, dec_b3):
    src = edge_index[0]
    dst = edge_index[1]

    perm = jnp.argsort(dst)
    dst_s = dst[perm]
    rec = jnp.concatenate(
        [edge_attr[perm], dst_s[:, None].astype(_f32),
         jnp.zeros((_E, 12), _f32)], axis=1)
    rec = jnp.pad(rec, ((0, _EPAD - _E), (0, 0)))
    srcp = jnp.pad(src[perm], (0, _EPAD - _E))
    dstp = jnp.pad(dst_s, (0, _EPAD - _E))
    bounds = jnp.searchsorted(dst_s, jnp.arange(_NW + 1, dtype=_i32) * _RPT)
    bounds = bounds.astype(_i32)
    lo_t = jnp.broadcast_to(bounds[:_NW, None], (_NW, 16))
    cnt_t = jnp.broadcast_to((bounds[1:] - bounds[:_NW])[:, None], (_NW, 16))
    nb = jnp.searchsorted(dst_s, jnp.arange(_N + 1, dtype=_i32))
    deg_arr = (nb[1:] - nb[:_N]).astype(_f32)[:, None]

    h = _enc_call(x, enc_w1, enc_b1.reshape(1, _H), enc_w2, enc_b2.reshape(1, _H))

    for l in range(_L):
        wa = msg_w1[l, :_H, :]
        wb = msg_w1[l, _H:2 * _H, :]
        wcb = jnp.concatenate([msg_w1[l, 2 * _H:, :], msg_b1[l][None, :]], axis=0)
        a_arr, b_arr = _ab_call(h, wa, wb)
        s_sum = _edge_call(a_arr, b_arr, rec, srcp, dstp, lo_t, cnt_t, wcb)
        h = _node_call(h, s_sum, deg_arr, params,
                       msg_w2[l], msg_b2[l].reshape(1, _H),
                       upd_w[l, :_H, :], upd_w[l, _H:, :],
                       upd_b[l].reshape(1, _H),
                       ln_g[l].reshape(1, _H), ln_b[l].reshape(1, _H),
                       film_w1[l], film_b1[l].reshape(1, _H),
                       film_w2[l], film_b2[l].reshape(1, 2 * _H))

    return _dec_call(h, dec_w1, dec_b1.reshape(1, _H),
                     dec_w2, dec_b2.reshape(1, _H // 2),
                     dec_w3, dec_b3.reshape(1, 8))
